# baseline (speedup 1.0000x reference)
import functools

import jax
import jax.numpy as jnp
from jax import lax
from jax.experimental import pallas as pl
from jax.experimental.pallas import tpu as pltpu
from jax.experimental.pallas import tpu_sc as plsc

_B, _T, _F = 16, 2048, 80
_B_TC = 12
_B_SC = _B - _B_TC
_BB = 4
_CHUNK_T = 256


@functools.partial(
    pl.kernel,
    out_type=[
        jax.ShapeDtypeStruct((_B_SC, _T, _F), jnp.float32),
        jax.ShapeDtypeStruct((_B,), jnp.int32),
    ],
    mesh=plsc.VectorSubcoreMesh(core_axis_name="c", subcore_axis_name="s"),
    scratch_types=[
        pltpu.VMEM((_CHUNK_T, _F), jnp.float32),
        pltpu.VMEM((_B,), jnp.int32),
    ],
)
def _sc_tail(wav_hbm, len_hbm, wav_out, len_out, buf, len_buf):
    c = lax.axis_index("c")
    s = lax.axis_index("s")
    wid = s * 2 + c
    b = wid // 8
    t0 = (wid % 8) * _CHUNK_T
    pltpu.sync_copy(wav_hbm.at[b, pl.ds(t0, _CHUNK_T)], buf)
    pltpu.sync_copy(buf, wav_out.at[b, pl.ds(t0, _CHUNK_T)])

    @pl.when(wid == 0)
    def _():
        pltpu.sync_copy(len_hbm, len_buf)
        pltpu.sync_copy(len_buf, len_out)


def _tc_copy_kernel(wav_ref, wav_out_ref):
    wav_out_ref[...] = wav_ref[...]


def kernel(wav_batch, lengths):
    lengths_i32 = jnp.asarray(lengths).astype(jnp.int32)
    sc_out, len_out = _sc_tail(
        jax.lax.slice_in_dim(wav_batch, _B_TC, _B, axis=0), lengths_i32
    )
    tc_out = pl.pallas_call(
        _tc_copy_kernel,
        grid=(_B_TC // _BB,),
        in_specs=[pl.BlockSpec((_BB, _T, _F), lambda i: (i, 0, 0))],
        out_specs=pl.BlockSpec((_BB, _T, _F), lambda i: (i, 0, 0)),
        out_shape=jax.ShapeDtypeStruct((_B_TC, _T, _F), wav_batch.dtype),
    )(jax.lax.slice_in_dim(wav_batch, 0, _B_TC, axis=0))
    wav_out = jnp.concatenate([tc_out, sc_out], axis=0)
    return wav_out, len_out


# speedup vs baseline: 1.2587x; 1.2587x over previous
import functools

import jax
import jax.numpy as jnp
from jax import lax
from jax.experimental import pallas as pl
from jax.experimental.pallas import tpu as pltpu
from jax.experimental.pallas import tpu_sc as plsc

_B, _T, _F = 16, 2048, 80
_HALF_T = _T // 2
_CHUNK_T = 256
_NCHUNK = _HALF_T // _CHUNK_T


@functools.partial(
    pl.kernel,
    out_type=[
        jax.ShapeDtypeStruct((_B, _T, _F), jnp.float32),
        jax.ShapeDtypeStruct((_B,), jnp.int32),
    ],
    mesh=plsc.VectorSubcoreMesh(core_axis_name="c", subcore_axis_name="s"),
    scratch_types=[
        pltpu.VMEM((_CHUNK_T, _F), jnp.float32),
        pltpu.VMEM((_CHUNK_T, _F), jnp.float32),
        pltpu.VMEM((_B,), jnp.int32),
        pltpu.SemaphoreType.DMA,
        pltpu.SemaphoreType.DMA,
        pltpu.SemaphoreType.DMA,
        pltpu.SemaphoreType.DMA,
    ],
)
def _sc_materialize(wav_hbm, len_hbm, wav_out, len_out,
                    buf0, buf1, len_buf, isem0, isem1, osem0, osem1):
    c = lax.axis_index("c")
    s = lax.axis_index("s")
    wid = s * 2 + c
    b = wid // 2
    t0 = (wid % 2) * _HALF_T

    bufs = (buf0, buf1)
    isems = (isem0, isem1)
    osems = (osem0, osem1)

    def in_cp(k):
        return pltpu.make_async_copy(
            wav_hbm.at[b, pl.ds(t0 + k * _CHUNK_T, _CHUNK_T)],
            bufs[k % 2], isems[k % 2])

    def out_cp(k):
        return pltpu.make_async_copy(
            bufs[k % 2],
            wav_out.at[b, pl.ds(t0 + k * _CHUNK_T, _CHUNK_T)],
            osems[k % 2])

    in_cp(0).start()
    in_cp(1).start()
    in_cp(0).wait()
    out_cp(0).start()
    in_cp(1).wait()
    out_cp(1).start()
    out_cp(0).wait()
    in_cp(2).start()
    out_cp(1).wait()
    in_cp(3).start()
    in_cp(2).wait()
    out_cp(2).start()
    in_cp(3).wait()
    out_cp(3).start()

    @pl.when(wid == 0)
    def _():
        pltpu.sync_copy(len_hbm, len_buf)
        pltpu.sync_copy(len_buf, len_out)

    out_cp(2).wait()
    out_cp(3).wait()


def kernel(wav_batch, lengths):
    lengths_i32 = jnp.asarray(lengths).astype(jnp.int32)
    wav_out, len_out = _sc_materialize(wav_batch, lengths_i32)
    return wav_out, len_out
